# baseline (speedup 1.0000x reference)
import jax
import jax.numpy as jnp
from jax import lax
from jax.experimental import pallas as pl
from jax.experimental.pallas import tpu as pltpu
from jax.experimental.pallas import tpu_sc as plsc

_B = 16384
_D = 64
_NREL = 100000
_NC, _NS, _L = 2, 16, 16
_NW = _NC * _NS
_RPW = _B // _NW
_TCW = 25
_S = 4
_IDS_PP = _S * 128
_FULL_TC = _NREL // 128
_PART_LO = _FULL_TC * 128
_DUMP = _B
_OUTROWS = _B + 128
_HCAP = _B + 128


def _p1_body(tabT_hbm, tailT_hbm, idx_hbm, out_hbm,
             stage_v, ids_v, hid_v, hb_v, pid_v, pb_v, rows_v, bidx_v, sem):
    wid = lax.axis_index("s") * _NC + lax.axis_index("c")
    lane = lax.iota(jnp.int32, _L)
    wlo = wid * (_TCW * 128)
    whi = jnp.minimum(wlo + _TCW * 128, _PART_LO)
    tc0 = wid * _TCW

    tot = jnp.zeros((), jnp.int32)
    for blk in range(8):
        pltpu.sync_copy(idx_hbm.at[pl.ds(blk * 2048, 2048)], ids_v)

        def scan(i, tot):
            v = ids_v[pl.ds(i * _L, _L)]
            b = blk * 2048 + i * _L + lane
            m = (v >= wlo) & (v < jnp.minimum(wlo + _TCW * 128, _NREL))
            plsc.store_compressed(hid_v.at[pl.ds(tot, _L)], v, mask=m)
            plsc.store_compressed(hb_v.at[pl.ds(tot, _L)], b, mask=m)
            return tot + plsc.all_reduce_population_count(m)[0]

        tot = lax.fori_loop(0, 128, scan, tot)

    trv = [jnp.full((_L,), j, jnp.int32) for j in range(8)]

    def one_pass(p, carry):
        plo = wlo + p * _IDS_PP
        phi = jnp.minimum(jnp.minimum(plo + _IDS_PP, whi), _NREL)
        is_part = p == 7
        plo = jnp.where(is_part, _PART_LO, plo)
        phi = jnp.where(is_part, jnp.minimum(wlo + _TCW * 128, _NREL), phi)

        @pl.when(jnp.logical_not(is_part))
        def _():
            cps = []
            for j in range(_S):
                tcj = jnp.minimum(jnp.minimum(tc0 + p * _S + j, tc0 + _TCW - 1),
                                  _FULL_TC - 1)
                for tr in range(8):
                    cps.append(pltpu.async_copy(
                        tabT_hbm.at[pl.ds(tr * 8, 8), pl.ds(tcj * 128, 128)],
                        stage_v.at[j, tr], sem))
            for c in cps:
                c.wait()

        @pl.when(is_part)
        def _():
            cps = []
            for tr in range(8):
                cps.append(pltpu.async_copy(
                    tailT_hbm.at[pl.ds(tr * 8, 8), pl.ds(0, 128)],
                    stage_v.at[0, tr], sem))
            for c in cps:
                c.wait()

        def rescan(g, np_):
            v = hid_v[pl.ds(g * _L, _L)]
            b = hb_v[pl.ds(g * _L, _L)]
            m = ((g * _L + lane) < tot) & (v >= plo) & (v < phi)
            plsc.store_compressed(pid_v.at[pl.ds(np_, _L)], v, mask=m)
            plsc.store_compressed(pb_v.at[pl.ds(np_, _L)], b, mask=m)
            return np_ + plsc.all_reduce_population_count(m)[0]

        np_ = lax.fori_loop(0, (tot + _L - 1) // _L, rescan,
                            jnp.zeros((), jnp.int32))
        fill_id = jnp.broadcast_to(plo, (_L,))
        fill_b = jnp.full((_L,), _DUMP, jnp.int32)
        for j in range(8):
            pid_v[pl.ds(np_ + j * _L, _L)] = fill_id
            pb_v[pl.ds(np_ + j * _L, _L)] = fill_b

        def batch(k, c2):
            for gi in range(8):
                off = k * 128 + gi * _L
                bv = pb_v[pl.ds(off, _L)]
                bidx_v[0, pl.ds(gi * _L, _L)] = bv
                idv = pid_v[pl.ds(off, _L)]
                tc = (idv - plo) >> 7
                col = idv & 127
                o = bv & 15
                rowi = gi * _L + lane
                for d in range(_D):
                    e = plsc.load_gather(
                        stage_v, [tc, trv[d >> 3], trv[d & 7], col])
                    plsc.store_scatter(rows_v, [rowi, o + d], e)
            pltpu.sync_copy(rows_v, out_hbm.at[bidx_v.at[0]])
            return c2

        lax.fori_loop(0, (np_ + 127) // 128, batch, jnp.zeros((), jnp.int32))
        return carry

    lax.fori_loop(0, 8, one_pass, jnp.zeros((), jnp.int32))


def _p2_body(srcT_hbm, trgT_hbm, rel_hbm, scores_hbm, parts_hbm,
             sT_v, tT_v, relc_v, scores_v, sq_v, sem):
    wid = lax.axis_index("s") * _NC + lax.axis_index("c")
    lane = lax.iota(jnp.int32, _L)
    sq = jnp.zeros((_L,), jnp.float32)
    for sp in range(2):
        b0 = wid * _RPW + sp * 256
        cps = [pltpu.async_copy(rel_hbm.at[pl.ds(b0, 256)], relc_v, sem)]
        tcb = b0 // 128
        for tcj in range(2):
            for tr in range(8):
                sl = (pl.ds(tr * 8, 8), pl.ds((tcb + tcj) * 128, 128))
                cps.append(pltpu.async_copy(srcT_hbm.at[sl[0], sl[1]],
                                            sT_v.at[tcj, tr], sem))
                cps.append(pltpu.async_copy(trgT_hbm.at[sl[0], sl[1]],
                                            tT_v.at[tcj, tr], sem))
        for c in cps:
            c.wait()

        def group(g, sq):
            rowv = g * _L + lane
            tcg = (g * _L) // 128
            off = (g * _L) % 128
            acc = jnp.zeros((_L,), jnp.float32)
            for d in range(_D):
                s = sT_v[tcg, d >> 3, d & 7, pl.ds(off, _L)]
                t = tT_v[tcg, d >> 3, d & 7, pl.ds(off, _L)]
                e = plsc.load_gather(relc_v, [rowv, lane + d])
                acc = acc + (s * t) * e
                sq = sq + (s * s + t * t + e * e)
            scores_v[pl.ds(sp * 256 + g * _L, _L)] = acc
            return sq

        sq = lax.fori_loop(0, 16, group, sq)
    sq_v[...] = sq
    pltpu.sync_copy(scores_v, scores_hbm.at[pl.ds(wid * _RPW, _RPW)])
    pltpu.sync_copy(sq_v, parts_hbm.at[pl.ds(wid * _L, _L)])


def _mesh():
    return plsc.VectorSubcoreMesh(core_axis_name="c", subcore_axis_name="s",
                                  num_cores=_NC, num_subcores=_NS)


_CP = pltpu.CompilerParams(needs_layout_passes=False, use_tc_tiling_on_sc=True)


@jax.jit
def _run(srcT, trgT, ids, tabT):
    p1 = pl.kernel(
        _p1_body,
        out_type=jax.ShapeDtypeStruct((_OUTROWS, 128), jnp.float32),
        mesh=_mesh(),
        compiler_params=_CP,
        scratch_types=[
            pltpu.VMEM((_S, 8, 8, 128), jnp.float32),
            pltpu.VMEM((2048,), jnp.int32),
            pltpu.VMEM((_HCAP,), jnp.int32),
            pltpu.VMEM((_HCAP,), jnp.int32),
            pltpu.VMEM((_HCAP,), jnp.int32),
            pltpu.VMEM((_HCAP,), jnp.int32),
            pltpu.VMEM((128, 128), jnp.float32),
            pltpu.VMEM((1, 128), jnp.int32),
            pltpu.SemaphoreType.DMA,
        ],
    )
    tailT = jnp.pad(tabT[:, _PART_LO:], ((0, 0), (0, 128 - (_NREL - _PART_LO))))
    rel_pad = p1(tabT, tailT, ids)
    p2 = pl.kernel(
        _p2_body,
        out_type=(jax.ShapeDtypeStruct((_B,), jnp.float32),
                  jax.ShapeDtypeStruct((_NW * _L,), jnp.float32)),
        mesh=_mesh(),
        compiler_params=_CP,
        scratch_types=[
            pltpu.VMEM((2, 8, 8, 128), jnp.float32),
            pltpu.VMEM((2, 8, 8, 128), jnp.float32),
            pltpu.VMEM((256, 128), jnp.float32),
            pltpu.VMEM((_RPW,), jnp.float32),
            pltpu.VMEM((_L,), jnp.float32),
            pltpu.SemaphoreType.DMA,
        ],
    )
    return p2(srcT, trgT, rel_pad)


def kernel(src_node_embs, trg_node_embs, rel_ids, relation_embeddings):
    scores, parts = _run(src_node_embs.T, trg_node_embs.T,
                         rel_ids.astype(jnp.int32), relation_embeddings.T)
    reg = jnp.sum(parts) * (1.0 / (_B * _D))
    return scores, reg


# speedup vs baseline: 5.8607x; 5.8607x over previous
import jax
import jax.numpy as jnp
from jax import lax
from jax.experimental import pallas as pl
from jax.experimental.pallas import tpu as pltpu
from jax.experimental.pallas import tpu_sc as plsc

_B = 16384
_D = 64
_NREL = 100000
_NC, _NS, _L = 2, 16, 16
_NW = _NC * _NS
_RPW = _B // _NW
_TCW = 25
_S = 4
_IDS_PP = _S * 128
_FULL_TC = _NREL // 128
_PART_LO = _FULL_TC * 128
_DUMP = _B
_OUTROWS = _B + _NW * 128
_HCAP = _B + 128
_BATCH = 32


def _p1_body(tabT_hbm, tailT_hbm, idx_hbm, out_hbm,
             stage_v, ids_v, hit_v, pass_v, rows_v, bidx_v, sem):
    wid = lax.axis_index("s") * _NC + lax.axis_index("c")
    lane = lax.iota(jnp.int32, _L)
    wlo = wid * (_TCW * 128)
    wspan = jnp.minimum(_TCW * 128, _NREL - wlo)
    tc0 = wid * _TCW

    tot = jnp.zeros((), jnp.int32)
    for blk in range(8):
        pltpu.sync_copy(idx_hbm.at[pl.ds(blk * 2048, 2048)], ids_v)

        def scan(i, tot):
            for u in range(4):
                c = i * 4 + u
                v = ids_v[pl.ds(c * _L, _L)] - wlo
                b = blk * 2048 + c * _L + lane
                m = (v >= 0) & (v < wspan)
                pk = (v << 15) | b
                plsc.store_compressed(hit_v.at[pl.ds(tot, _L)], pk, mask=m)
                tot = tot + plsc.all_reduce_population_count(m)[0]
            return tot

        tot = lax.fori_loop(0, 32, scan, tot)

    def one_pass(p, carry):
        is_part = p == 7
        rlo = jnp.where(is_part, _PART_LO - wlo, p * _IDS_PP)
        rhi = jnp.minimum(jnp.where(is_part, wspan, rlo + _IDS_PP), wspan)

        cps = []

        @pl.when(jnp.logical_not(is_part))
        def _():
            dps = []
            for j in range(_S):
                tcj = jnp.minimum(jnp.minimum(tc0 + p * _S + j, tc0 + _TCW - 1),
                                  _FULL_TC - 1)
                for tr in range(8):
                    dps.append(pltpu.async_copy(
                        tabT_hbm.at[pl.ds(tr * 8, 8), pl.ds(tcj * 128, 128)],
                        stage_v.at[pl.ds(j * 64 + tr * 8, 8)], sem))
            for c in dps:
                c.wait()

        @pl.when(is_part)
        def _():
            dps = []
            for tr in range(8):
                dps.append(pltpu.async_copy(
                    tailT_hbm.at[pl.ds(tr * 8, 8), pl.ds(0, 128)],
                    stage_v.at[pl.ds(tr * 8, 8)], sem))
            for c in dps:
                c.wait()

        def rescan(g, np_):
            pk = hit_v[pl.ds(g * _L, _L)]
            v = pk >> 15
            m = ((g * _L + lane) < tot) & (v >= rlo) & (v < rhi)
            plsc.store_compressed(pass_v.at[pl.ds(np_, _L)], pk, mask=m)
            return np_ + plsc.all_reduce_population_count(m)[0]

        np_ = lax.fori_loop(0, (tot + _L - 1) // _L, rescan,
                            jnp.zeros((), jnp.int32))
        for j in range(_BATCH // _L):
            fill = (rlo << 15) | (_DUMP + wid * 128 + j * _L + lane)
            pass_v[pl.ds(np_ + j * _L, _L)] = fill

        def batch(k, c2):
            for gi in range(_BATCH // _L):
                pk = pass_v[pl.ds(k * _BATCH + gi * _L, _L)]
                v = pk >> 15
                b = pk & 32767
                bidx_v[0, pl.ds(gi * _L, _L)] = b
                tcs = ((v - rlo) >> 7) << 6
                col = v & 127
                o = b & 15
                rowi = gi * _L + lane
                for d in range(_D):
                    e = plsc.load_gather(stage_v, [tcs + d, col])
                    plsc.store_scatter(rows_v, [rowi, o + d], e)
            pltpu.sync_copy(rows_v, out_hbm.at[bidx_v.at[0]])
            return c2

        lax.fori_loop(0, (np_ + _BATCH - 1) // _BATCH, batch,
                      jnp.zeros((), jnp.int32))
        return carry

    lax.fori_loop(0, 8, one_pass, jnp.zeros((), jnp.int32))


def _p2_body(srcT_hbm, trgT_hbm, rel_hbm, scores_hbm, parts_hbm,
             sT_v, tT_v, relc_v, scores_v, sq_v, sem):
    wid = lax.axis_index("s") * _NC + lax.axis_index("c")
    lane = lax.iota(jnp.int32, _L)
    sq = jnp.zeros((_L,), jnp.float32)
    nsp = _RPW // 128

    def fire(sp, buf):
        b0 = wid * _RPW + sp * 128
        cps = [pltpu.async_copy(rel_hbm.at[pl.ds(b0, 128)], relc_v.at[buf],
                                sem)]
        tcb = b0 // 128
        for tr in range(8):
            sl = (pl.ds(tr * 8, 8), pl.ds(tcb * 128, 128))
            cps.append(pltpu.async_copy(srcT_hbm.at[sl[0], sl[1]],
                                        sT_v.at[buf, tr], sem))
            cps.append(pltpu.async_copy(trgT_hbm.at[sl[0], sl[1]],
                                        tT_v.at[buf, tr], sem))
        return cps

    pend = fire(0, 0)
    for sp in range(nsp):
        buf = sp % 2
        for c in pend:
            c.wait()
        pend = fire(sp + 1, 1 - buf) if sp + 1 < nsp else []

        def group(g, sq):
            rowv = g * _L + lane
            acc = jnp.zeros((_L,), jnp.float32)
            off = g * _L
            for d in range(_D):
                s = sT_v[buf, d >> 3, d & 7, pl.ds(off, _L)]
                t = tT_v[buf, d >> 3, d & 7, pl.ds(off, _L)]
                e = plsc.load_gather(relc_v.at[buf], [rowv, lane + d])
                acc = acc + (s * t) * e
                sq = sq + (s * s + t * t + e * e)
            scores_v[pl.ds(sp * 128 + g * _L, _L)] = acc
            return sq

        sq = lax.fori_loop(0, 8, group, sq)
    sq_v[...] = sq
    pltpu.sync_copy(scores_v, scores_hbm.at[pl.ds(wid * _RPW, _RPW)])
    pltpu.sync_copy(sq_v, parts_hbm.at[pl.ds(wid * _L, _L)])


def _mesh():
    return plsc.VectorSubcoreMesh(core_axis_name="c", subcore_axis_name="s",
                                  num_cores=_NC, num_subcores=_NS)


_CP = pltpu.CompilerParams(needs_layout_passes=False, use_tc_tiling_on_sc=True)


@jax.jit
def _run(srcT, trgT, ids, tabT):
    p1 = pl.kernel(
        _p1_body,
        out_type=jax.ShapeDtypeStruct((_OUTROWS, 128), jnp.float32),
        mesh=_mesh(),
        compiler_params=_CP,
        scratch_types=[
            pltpu.VMEM((_S * 64, 128), jnp.float32),
            pltpu.VMEM((2048,), jnp.int32),
            pltpu.VMEM((_HCAP,), jnp.int32),
            pltpu.VMEM((_HCAP,), jnp.int32),
            pltpu.VMEM((_BATCH, 128), jnp.float32),
            pltpu.VMEM((1, _BATCH), jnp.int32),
            pltpu.SemaphoreType.DMA,
        ],
    )
    tailT = jnp.pad(tabT[:, _PART_LO:], ((0, 0), (0, 128 - (_NREL - _PART_LO))))
    rel_pad = p1(tabT, tailT, ids)
    p2 = pl.kernel(
        _p2_body,
        out_type=(jax.ShapeDtypeStruct((_B,), jnp.float32),
                  jax.ShapeDtypeStruct((_NW * _L,), jnp.float32)),
        mesh=_mesh(),
        compiler_params=_CP,
        scratch_types=[
            pltpu.VMEM((2, 8, 8, 128), jnp.float32),
            pltpu.VMEM((2, 8, 8, 128), jnp.float32),
            pltpu.VMEM((2, 128, 128), jnp.float32),
            pltpu.VMEM((_RPW,), jnp.float32),
            pltpu.VMEM((_L,), jnp.float32),
            pltpu.SemaphoreType.DMA,
        ],
    )
    return p2(srcT, trgT, rel_pad)


def kernel(src_node_embs, trg_node_embs, rel_ids, relation_embeddings):
    scores, parts = _run(src_node_embs.T, trg_node_embs.T,
                         rel_ids.astype(jnp.int32), relation_embeddings.T)
    reg = jnp.sum(parts) * (1.0 / (_B * _D))
    return scores, reg


# speedup vs baseline: 6.2046x; 1.0587x over previous
import jax
import jax.numpy as jnp
from jax import lax
from jax.experimental import pallas as pl
from jax.experimental.pallas import tpu as pltpu
from jax.experimental.pallas import tpu_sc as plsc

_B = 16384
_D = 64
_NREL = 100000
_NC, _NS, _L = 2, 16, 16
_NW = _NC * _NS
_RPW = _B // _NW
_TCW = 25
_S = 4
_IDS_PP = _S * 128
_FULL_TC = _NREL // 128
_PART_LO = _FULL_TC * 128
_DUMP = _B
_OUTROWS = _B + _NW * 128
_HCAP = _B + 128
_BATCH = 32


def _p1_body(tabT_hbm, tailT_hbm, idx_hbm, out_hbm,
             stage_v, ids_v, hit_v, pass_v, rows_v, bidx_v, sem):
    wid = lax.axis_index("s") * _NC + lax.axis_index("c")
    lane = lax.iota(jnp.int32, _L)
    wlo = wid * (_TCW * 128)
    wspan = jnp.minimum(_TCW * 128, _NREL - wlo)
    tc0 = wid * _TCW

    tot = jnp.zeros((), jnp.int32)
    for blk in range(8):
        pltpu.sync_copy(idx_hbm.at[pl.ds(blk * 2048, 2048)], ids_v)

        def scan(i, tot):
            for u in range(4):
                c = i * 4 + u
                v = ids_v[pl.ds(c * _L, _L)] - wlo
                b = blk * 2048 + c * _L + lane
                m = (v >= 0) & (v < wspan)
                pk = (v << 15) | b
                plsc.store_compressed(hit_v.at[pl.ds(tot, _L)], pk, mask=m)
                tot = tot + plsc.all_reduce_population_count(m)[0]
            return tot

        tot = lax.fori_loop(0, 32, scan, tot)

    def fire(p, buf):
        dst = stage_v.at[buf]
        if p == 7:
            return [pltpu.async_copy(
                tailT_hbm.at[pl.ds(tr * 8, 8), pl.ds(0, 128)],
                dst.at[pl.ds(tr * 8, 8)], sem) for tr in range(8)]
        dps = []
        for j in range(_S):
            tcj = jnp.minimum(jnp.minimum(tc0 + p * _S + j, tc0 + _TCW - 1),
                              _FULL_TC - 1)
            for tr in range(8):
                dps.append(pltpu.async_copy(
                    tabT_hbm.at[pl.ds(tr * 8, 8), pl.ds(tcj * 128, 128)],
                    dst.at[pl.ds(j * 64 + tr * 8, 8)], sem))
        return dps

    pend = fire(0, 0)
    for p in range(8):
        buf = p & 1
        is_part = p == 7
        if is_part:
            rlo = _PART_LO - wlo
            rhi = wspan
        else:
            rlo = p * _IDS_PP
            rhi = jnp.minimum(rlo + _IDS_PP, wspan)
        rlo_v = jnp.broadcast_to(rlo, ())

        def rescan(g, np_, rlo=rlo, rhi=rhi):
            pk = hit_v[pl.ds(g * _L, _L)]
            v = pk >> 15
            m = ((g * _L + lane) < tot) & (v >= rlo) & (v < rhi)
            plsc.store_compressed(pass_v.at[pl.ds(np_, _L)], pk, mask=m)
            return np_ + plsc.all_reduce_population_count(m)[0]

        np_ = lax.fori_loop(0, (tot + _L - 1) // _L, rescan,
                            jnp.zeros((), jnp.int32))
        for j in range(_BATCH // _L):
            fill = (rlo_v << 15) | (_DUMP + wid * 128 + j * _L + lane)
            pass_v[pl.ds(np_ + j * _L, _L)] = fill

        for c in pend:
            c.wait()
        pend = fire(p + 1, 1 - buf) if p + 1 < 8 else []
        stg = stage_v.at[buf]

        def batch(k, c2, rlo=rlo, stg=stg):
            for gi in range(_BATCH // _L):
                pk = pass_v[pl.ds(k * _BATCH + gi * _L, _L)]
                v = pk >> 15
                b = pk & 32767
                bidx_v[0, pl.ds(gi * _L, _L)] = b
                tcs = ((v - rlo) >> 7) << 6
                col = v & 127
                o = b & 15
                rowi = gi * _L + lane
                for d in range(_D):
                    e = plsc.load_gather(stg, [tcs + d, col])
                    plsc.store_scatter(rows_v, [rowi, o + d], e)
            pltpu.sync_copy(rows_v, out_hbm.at[bidx_v.at[0]])
            return c2

        lax.fori_loop(0, (np_ + _BATCH - 1) // _BATCH, batch,
                      jnp.zeros((), jnp.int32))


def _p2_body(srcT_hbm, trgT_hbm, rel_hbm, scores_hbm, parts_hbm,
             sT_v, tT_v, relc_v, scores_v, sq_v, sem):
    wid = lax.axis_index("s") * _NC + lax.axis_index("c")
    lane = lax.iota(jnp.int32, _L)
    sq = jnp.zeros((_L,), jnp.float32)
    nsp = _RPW // 128

    def fire(sp, buf):
        b0 = wid * _RPW + sp * 128
        cps = [pltpu.async_copy(rel_hbm.at[pl.ds(b0, 128)], relc_v.at[buf],
                                sem)]
        tcb = b0 // 128
        for tr in range(8):
            sl = (pl.ds(tr * 8, 8), pl.ds(tcb * 128, 128))
            cps.append(pltpu.async_copy(srcT_hbm.at[sl[0], sl[1]],
                                        sT_v.at[buf, tr], sem))
            cps.append(pltpu.async_copy(trgT_hbm.at[sl[0], sl[1]],
                                        tT_v.at[buf, tr], sem))
        return cps

    pend = fire(0, 0)
    for sp in range(nsp):
        buf = sp % 2
        for c in pend:
            c.wait()
        pend = fire(sp + 1, 1 - buf) if sp + 1 < nsp else []

        def group(g, sq):
            rowv = g * _L + lane
            acc = jnp.zeros((_L,), jnp.float32)
            off = g * _L
            for d in range(_D):
                s = sT_v[buf, d >> 3, d & 7, pl.ds(off, _L)]
                t = tT_v[buf, d >> 3, d & 7, pl.ds(off, _L)]
                e = plsc.load_gather(relc_v.at[buf], [rowv, lane + d])
                acc = acc + (s * t) * e
                sq = sq + (s * s + t * t + e * e)
            scores_v[pl.ds(sp * 128 + g * _L, _L)] = acc
            return sq

        sq = lax.fori_loop(0, 8, group, sq)
    sq_v[...] = sq
    pltpu.sync_copy(scores_v, scores_hbm.at[pl.ds(wid * _RPW, _RPW)])
    pltpu.sync_copy(sq_v, parts_hbm.at[pl.ds(wid * _L, _L)])


def _mesh():
    return plsc.VectorSubcoreMesh(core_axis_name="c", subcore_axis_name="s",
                                  num_cores=_NC, num_subcores=_NS)


_CP = pltpu.CompilerParams(needs_layout_passes=False, use_tc_tiling_on_sc=True)


@jax.jit
def _run(srcT, trgT, ids, tabT):
    p1 = pl.kernel(
        _p1_body,
        out_type=jax.ShapeDtypeStruct((_OUTROWS, 128), jnp.float32),
        mesh=_mesh(),
        compiler_params=_CP,
        scratch_types=[
            pltpu.VMEM((2, _S * 64, 128), jnp.float32),
            pltpu.VMEM((2048,), jnp.int32),
            pltpu.VMEM((_HCAP,), jnp.int32),
            pltpu.VMEM((_HCAP,), jnp.int32),
            pltpu.VMEM((_BATCH, 128), jnp.float32),
            pltpu.VMEM((1, _BATCH), jnp.int32),
            pltpu.SemaphoreType.DMA,
        ],
    )
    tailT = jnp.pad(tabT[:, _PART_LO:], ((0, 0), (0, 128 - (_NREL - _PART_LO))))
    rel_pad = p1(tabT, tailT, ids)
    p2 = pl.kernel(
        _p2_body,
        out_type=(jax.ShapeDtypeStruct((_B,), jnp.float32),
                  jax.ShapeDtypeStruct((_NW * _L,), jnp.float32)),
        mesh=_mesh(),
        compiler_params=_CP,
        scratch_types=[
            pltpu.VMEM((2, 8, 8, 128), jnp.float32),
            pltpu.VMEM((2, 8, 8, 128), jnp.float32),
            pltpu.VMEM((2, 128, 128), jnp.float32),
            pltpu.VMEM((_RPW,), jnp.float32),
            pltpu.VMEM((_L,), jnp.float32),
            pltpu.SemaphoreType.DMA,
        ],
    )
    return p2(srcT, trgT, rel_pad)


def kernel(src_node_embs, trg_node_embs, rel_ids, relation_embeddings):
    scores, parts = _run(src_node_embs.T, trg_node_embs.T,
                         rel_ids.astype(jnp.int32), relation_embeddings.T)
    reg = jnp.sum(parts) * (1.0 / (_B * _D))
    return scores, reg
